# baseline (speedup 1.0000x reference)
import jax
import jax.numpy as jnp
from jax import lax
from jax.experimental import pallas as pl
from jax.experimental.pallas import tpu as pltpu
from jax.experimental.pallas import tpu_sc as plsc

VOCAB = 1000000
EMBED_DIM = 32
BATCH = 16384
SEQ = 200

_NW = 32
_W = BATCH // _NW
_OB = 4 * _W * EMBED_DIM // 4096


def _body(enc_t, table, out5,
          idx0, idx1, rows0, rows1, ob0, ob1,
          isem0, isem1, gsem0, gsem1, osem0, osem1):
    w = lax.axis_index("s") * 2 + lax.axis_index("c")
    b0 = w * _W
    idx_v = (idx0, idx1)
    rows_v = (rows0, rows1)
    obuf = (ob0, ob1)
    isem = (isem0, isem1)
    gsem = (gsem0, gsem1)
    osem = (osem0, osem1)
    iota16 = lax.iota(jnp.int32, 16)

    def idx_start(s, b):
        pltpu.async_copy(enc_t.at[s, pl.ds(b0, _W)], idx_v[b], isem[b])

    def idx_wait(b):
        pltpu.make_async_copy(
            enc_t.at[0, pl.ds(b0, _W)], idx_v[b], isem[b]).wait()

    def gather_start(b):
        pltpu.async_copy(table.at[idx_v[b]], rows_v[b], gsem[b])

    def gather_wait(b):
        pltpu.make_async_copy(table.at[idx_v[b]], rows_v[b], gsem[b]).wait()

    def out_start(s, b):
        for r2 in range(4):
            for c2 in range(4):
                pltpu.async_copy(
                    obuf[b].at[pl.ds(8 * r2, 8), pl.ds(128 * c2, 128)],
                    out5.at[s, r2, 4 * w + c2], osem[b])

    def out_wait(b):
        for r2 in range(4):
            for c2 in range(4):
                pltpu.make_async_copy(
                    obuf[b].at[pl.ds(8 * r2, 8), pl.ds(128 * c2, 128)],
                    out5.at[0, r2, 4 * w + c2], osem[b]).wait()

    def transpose(b):
        rows = rows_v[b]
        ob = obuf[b]

        def jb_body(jb, carry):
            rowv = iota16 + jb * 16
            for kk in range(32):
                colv = iota16 ^ kk
                v = plsc.load_gather(rows, [rowv, colv])
                plsc.store_scatter(ob, [colv, rowv], v)
            return carry

        lax.fori_loop(0, _W // 16, jb_body, 0)

    def step(s, b):
        bp = 1 - b

        @pl.when(s + 1 < SEQ)
        def _next_gather():
            idx_wait(bp)
            gather_start(bp)

        gather_wait(b)

        @pl.when(s + 2 < SEQ)
        def _prefetch_idx():
            idx_start(s + 2, b)

        @pl.when(s >= 2)
        def _reclaim_obuf():
            out_wait(b)

        transpose(b)
        out_start(s, b)

    idx_start(0, 0)
    idx_start(1, 1)
    idx_wait(0)
    gather_start(0)

    def pair(p, carry):
        step(2 * p, 0)
        step(2 * p + 1, 1)
        return carry

    lax.fori_loop(0, SEQ // 2, pair, 0)

    out_wait(0)
    out_wait(1)


@jax.jit
def kernel(enc, enc_mask, table):
    enc_t = enc.T
    mesh = plsc.VectorSubcoreMesh(core_axis_name="c", subcore_axis_name="s")
    gather = pl.kernel(
        _body,
        mesh=mesh,
        out_type=jax.ShapeDtypeStruct((SEQ, 4, 128, 8, 128), jnp.float32),
        scratch_types=[
            pltpu.VMEM((_W,), jnp.int32),
            pltpu.VMEM((_W,), jnp.int32),
            pltpu.VMEM((_W, EMBED_DIM), jnp.float32),
            pltpu.VMEM((_W, EMBED_DIM), jnp.float32),
            pltpu.VMEM((EMBED_DIM, _W), jnp.float32),
            pltpu.VMEM((EMBED_DIM, _W), jnp.float32),
            pltpu.SemaphoreType.DMA,
            pltpu.SemaphoreType.DMA,
            pltpu.SemaphoreType.DMA,
            pltpu.SemaphoreType.DMA,
            pltpu.SemaphoreType.DMA,
            pltpu.SemaphoreType.DMA,
        ],
        compiler_params=pltpu.CompilerParams(
            use_tc_tiling_on_sc=False, needs_layout_passes=False),
    )
    o5 = gather(enc_t, table)
    dec = o5.transpose(2, 4, 0, 1, 3).reshape(BATCH, SEQ, EMBED_DIM)
    return (dec, enc_mask)


# speedup vs baseline: 1.0022x; 1.0022x over previous
import jax
import jax.numpy as jnp
from jax import lax
from jax.experimental import pallas as pl
from jax.experimental.pallas import tpu as pltpu
from jax.experimental.pallas import tpu_sc as plsc

VOCAB = 1000000
EMBED_DIM = 32
BATCH = 16384
SEQ = 200

_NW = 32
_W = BATCH // _NW
_OB = 4 * _W * EMBED_DIM // 4096


def _body(enc4, table, out5,
          idx0, idx1, rows0, rows1, ob0, ob1,
          isem0, isem1, gsem0, gsem1, osem0, osem1):
    w = lax.axis_index("s") * 2 + lax.axis_index("c")
    idx_v = (idx0, idx1)
    rows_v = (rows0, rows1)
    obuf = (ob0, ob1)
    isem = (isem0, isem1)
    gsem = (gsem0, gsem1)
    osem = (osem0, osem1)
    iota16 = lax.iota(jnp.int32, 16)

    def idx_start(s, b):
        for c2 in range(4):
            pltpu.async_copy(
                enc4.at[s >> 3, 4 * w + c2, s & 7],
                idx_v[b].at[pl.ds(128 * c2, 128)], isem[b])

    def idx_wait(b):
        for c2 in range(4):
            pltpu.make_async_copy(
                enc4.at[0, 0, 0],
                idx_v[b].at[pl.ds(128 * c2, 128)], isem[b]).wait()

    def gather_start(b):
        pltpu.async_copy(table.at[idx_v[b]], rows_v[b], gsem[b])

    def gather_wait(b):
        pltpu.make_async_copy(table.at[idx_v[b]], rows_v[b], gsem[b]).wait()

    def out_start(s, b):
        for r2 in range(4):
            for c2 in range(4):
                pltpu.async_copy(
                    obuf[b].at[pl.ds(8 * r2, 8), pl.ds(128 * c2, 128)],
                    out5.at[s, r2, 4 * w + c2], osem[b])

    def out_wait(b):
        for r2 in range(4):
            for c2 in range(4):
                pltpu.make_async_copy(
                    obuf[b].at[pl.ds(8 * r2, 8), pl.ds(128 * c2, 128)],
                    out5.at[0, r2, 4 * w + c2], osem[b]).wait()

    def transpose(b):
        rows = rows_v[b]
        ob = obuf[b]

        def jb_body(jb, carry):
            rowv = iota16 + jb * 16
            for kk in range(32):
                colv = iota16 ^ kk
                v = plsc.load_gather(rows, [rowv, colv])
                plsc.store_scatter(ob, [colv, rowv], v)
            return carry

        lax.fori_loop(0, _W // 16, jb_body, 0)

    def step(s, b):
        bp = 1 - b

        @pl.when(s + 1 < SEQ)
        def _next_gather():
            idx_wait(bp)
            gather_start(bp)

        gather_wait(b)

        @pl.when(s + 2 < SEQ)
        def _prefetch_idx():
            idx_start(s + 2, b)

        @pl.when(s >= 2)
        def _reclaim_obuf():
            out_wait(b)

        transpose(b)
        out_start(s, b)

    idx_start(0, 0)
    idx_start(1, 1)
    idx_wait(0)
    gather_start(0)

    def pair(p, carry):
        step(2 * p, 0)
        step(2 * p + 1, 1)
        return carry

    lax.fori_loop(0, SEQ // 2, pair, 0)

    out_wait(0)
    out_wait(1)


@jax.jit
def kernel(enc, enc_mask, table):
    enc4 = enc.T.reshape(25, 8, 128, 128).transpose(0, 2, 1, 3)
    mesh = plsc.VectorSubcoreMesh(core_axis_name="c", subcore_axis_name="s")
    gather = pl.kernel(
        _body,
        mesh=mesh,
        out_type=jax.ShapeDtypeStruct((SEQ, 4, 128, 8, 128), jnp.float32),
        scratch_types=[
            pltpu.VMEM((_W,), jnp.int32),
            pltpu.VMEM((_W,), jnp.int32),
            pltpu.VMEM((_W, EMBED_DIM), jnp.float32),
            pltpu.VMEM((_W, EMBED_DIM), jnp.float32),
            pltpu.VMEM((EMBED_DIM, _W), jnp.float32),
            pltpu.VMEM((EMBED_DIM, _W), jnp.float32),
            pltpu.SemaphoreType.DMA,
            pltpu.SemaphoreType.DMA,
            pltpu.SemaphoreType.DMA,
            pltpu.SemaphoreType.DMA,
            pltpu.SemaphoreType.DMA,
            pltpu.SemaphoreType.DMA,
        ],
        compiler_params=pltpu.CompilerParams(
            use_tc_tiling_on_sc=False, needs_layout_passes=False),
    )
    o5 = gather(enc4, table)
    dec = o5.transpose(2, 4, 0, 1, 3).reshape(BATCH, SEQ, EMBED_DIM)
    return (dec, enc_mask)


# speedup vs baseline: 1.0315x; 1.0293x over previous
import jax
import jax.numpy as jnp
from jax import lax
from jax.experimental import pallas as pl
from jax.experimental.pallas import tpu as pltpu
from jax.experimental.pallas import tpu_sc as plsc

VOCAB = 1000000
EMBED_DIM = 32
BATCH = 16384
SEQ = 200

_NW = 32
_W = BATCH // _NW
_OB = 4 * _W * EMBED_DIM // 4096


def _body(enc4, table, out3,
          idx0, idx1, rows0, rows1, ob0, ob1,
          isem0, isem1, gsem0, gsem1, osem0, osem1):
    w = lax.axis_index("s") * 2 + lax.axis_index("c")
    k0 = w * (_W * 8)
    idx_v = (idx0, idx1)
    rows_v = (rows0, rows1)
    obuf = (ob0, ob1)
    isem = (isem0, isem1)
    gsem = (gsem0, gsem1)
    osem = (osem0, osem1)
    iota16 = lax.iota(jnp.int32, 16)

    def idx_start(s, b):
        for c2 in range(4):
            pltpu.async_copy(
                enc4.at[s >> 3, 4 * w + c2, s & 7],
                idx_v[b].at[pl.ds(128 * c2, 128)], isem[b])

    def idx_wait(b):
        for c2 in range(4):
            pltpu.make_async_copy(
                enc4.at[0, 0, 0],
                idx_v[b].at[pl.ds(128 * c2, 128)], isem[b]).wait()

    def gather_start(b):
        pltpu.async_copy(table.at[idx_v[b]], rows_v[b], gsem[b])

    def gather_wait(b):
        pltpu.make_async_copy(table.at[idx_v[b]], rows_v[b], gsem[b]).wait()

    def out_start(s, b):
        for r2 in range(4):
            pltpu.async_copy(
                obuf[b].at[pl.ds(r2 * 4096, 4096)],
                out3.at[s, r2, pl.ds(k0, 4096)], osem[b])

    def out_wait(b):
        for r2 in range(4):
            pltpu.make_async_copy(
                obuf[b].at[pl.ds(r2 * 4096, 4096)],
                out3.at[0, r2, pl.ds(k0, 4096)], osem[b]).wait()

    colv = [iota16 ^ k for k in range(16)]
    sctr = [((iota16 ^ k) >> 3) * 4096 + ((iota16 ^ k) & 7) * 128 + iota16
            for k in range(16)]

    def transpose(b):
        rows = rows_v[b]
        ob = obuf[b]

        def jb_body(jb, carry):
            rowv = iota16 + jb * 16
            base_s = (jb >> 3) * 1024 + (jb & 7) * 16
            for eb in range(2):
                for k in range(16):
                    v = plsc.load_gather(rows, [rowv, colv[k] + 16 * eb])
                    plsc.store_scatter(
                        ob, [sctr[k] + (base_s + 8192 * eb)], v)
            return carry

        lax.fori_loop(0, _W // 16, jb_body, 0)

    def step(s, b):
        bp = 1 - b

        @pl.when(s + 1 < SEQ)
        def _next_gather():
            idx_wait(bp)
            gather_start(bp)

        gather_wait(b)

        @pl.when(s + 2 < SEQ)
        def _prefetch_idx():
            idx_start(s + 2, b)

        @pl.when(s >= 2)
        def _reclaim_obuf():
            out_wait(b)

        transpose(b)
        out_start(s, b)

    idx_start(0, 0)
    idx_start(1, 1)
    idx_wait(0)
    gather_start(0)

    def pair(p, carry):
        step(2 * p, 0)
        step(2 * p + 1, 1)
        return carry

    lax.fori_loop(0, SEQ // 2, pair, 0)

    out_wait(0)
    out_wait(1)


@jax.jit
def kernel(enc, enc_mask, table):
    enc4 = enc.T.reshape(25, 8, 128, 128).transpose(0, 2, 1, 3)
    mesh = plsc.VectorSubcoreMesh(core_axis_name="c", subcore_axis_name="s")
    gather = pl.kernel(
        _body,
        mesh=mesh,
        out_type=jax.ShapeDtypeStruct((SEQ, 4, BATCH * 8), jnp.float32),
        scratch_types=[
            pltpu.VMEM((_W,), jnp.int32),
            pltpu.VMEM((_W,), jnp.int32),
            pltpu.VMEM((_W, EMBED_DIM), jnp.float32),
            pltpu.VMEM((_W, EMBED_DIM), jnp.float32),
            pltpu.VMEM((_W * EMBED_DIM,), jnp.float32),
            pltpu.VMEM((_W * EMBED_DIM,), jnp.float32),
            pltpu.SemaphoreType.DMA,
            pltpu.SemaphoreType.DMA,
            pltpu.SemaphoreType.DMA,
            pltpu.SemaphoreType.DMA,
            pltpu.SemaphoreType.DMA,
            pltpu.SemaphoreType.DMA,
        ],
        compiler_params=pltpu.CompilerParams(
            use_tc_tiling_on_sc=False, needs_layout_passes=False),
    )
    o3 = gather(enc4, table)
    dec = (o3.reshape(SEQ, 4, 128, 8, 128)
              .transpose(2, 4, 0, 1, 3)
              .reshape(BATCH, SEQ, EMBED_DIM))
    return (dec, enc_mask)


# speedup vs baseline: 1.6961x; 1.6442x over previous
import jax
import jax.numpy as jnp
from jax import lax
from jax.experimental import pallas as pl
from jax.experimental.pallas import tpu as pltpu
from jax.experimental.pallas import tpu_sc as plsc

VOCAB = 1000000
EMBED_DIM = 32
BATCH = 16384
SEQ = 200

_NW = 32
_W = BATCH // _NW
_OB = 4 * _W * EMBED_DIM // 4096


def _body(enc4, table, out3,
          idx0, idx1, rows0, rows1, ob0, ob1,
          isem0, isem1, gsem0, gsem1, osem0, osem1):
    w = lax.axis_index("s") * 2 + lax.axis_index("c")
    k0 = w * (_W * 8)
    idx_v = (idx0, idx1)
    rows_v = (rows0, rows1)
    obuf = (ob0, ob1)
    isem = (isem0, isem1)
    gsem = (gsem0, gsem1)
    osem = (osem0, osem1)
    iota16 = lax.iota(jnp.int32, 16)

    def idx_start(s, b):
        for c2 in range(4):
            pltpu.async_copy(
                enc4.at[s >> 3, 4 * w + c2, s & 7],
                idx_v[b].at[pl.ds(128 * c2, 128)], isem[b])

    def idx_wait(b):
        for c2 in range(4):
            pltpu.make_async_copy(
                enc4.at[0, 0, 0],
                idx_v[b].at[pl.ds(128 * c2, 128)], isem[b]).wait()

    def gather_start(b):
        pltpu.async_copy(table.at[idx_v[b]], rows_v[b], gsem[b])

    def gather_wait(b):
        pltpu.make_async_copy(table.at[idx_v[b]], rows_v[b], gsem[b]).wait()

    def out_start(s, b):
        for r2 in range(4):
            pltpu.async_copy(
                obuf[b].at[pl.ds(r2 * 4096, 4096)],
                out3.at[s, r2, pl.ds(k0, 4096)], osem[b])

    def out_wait(b):
        for r2 in range(4):
            pltpu.make_async_copy(
                obuf[b].at[pl.ds(r2 * 4096, 4096)],
                out3.at[0, r2, pl.ds(k0, 4096)], osem[b]).wait()

    colv = [iota16 ^ k for k in range(16)]
    sctr = [((iota16 ^ k) >> 3) * 4096 + ((iota16 ^ k) & 7) * 128 + iota16
            for k in range(16)]

    def transpose(b):
        rows = rows_v[b]
        ob = obuf[b]

        @plsc.parallel_loop(0, _W // 16, unroll=2)
        def _jb_body(jb):
            rowv = iota16 + jb * 16
            base_s = (jb >> 3) * 1024 + (jb & 7) * 16
            for eb in range(2):
                for k in range(16):
                    v = plsc.load_gather(rows, [rowv, colv[k] + 16 * eb])
                    plsc.store_scatter(
                        ob, [sctr[k] + (base_s + 8192 * eb)], v)

    def step(s, b):
        bp = 1 - b

        @pl.when(s + 1 < SEQ)
        def _next_gather():
            idx_wait(bp)
            gather_start(bp)

        gather_wait(b)

        @pl.when(s + 2 < SEQ)
        def _prefetch_idx():
            idx_start(s + 2, b)

        @pl.when(s >= 2)
        def _reclaim_obuf():
            out_wait(b)

        transpose(b)
        out_start(s, b)

    idx_start(0, 0)
    idx_start(1, 1)
    idx_wait(0)
    gather_start(0)

    def pair(p, carry):
        step(2 * p, 0)
        step(2 * p + 1, 1)
        return carry

    lax.fori_loop(0, SEQ // 2, pair, 0)

    out_wait(0)
    out_wait(1)


@jax.jit
def kernel(enc, enc_mask, table):
    enc4 = enc.T.reshape(25, 8, 128, 128).transpose(0, 2, 1, 3)
    mesh = plsc.VectorSubcoreMesh(core_axis_name="c", subcore_axis_name="s")
    gather = pl.kernel(
        _body,
        mesh=mesh,
        out_type=jax.ShapeDtypeStruct((SEQ, 4, BATCH * 8), jnp.float32),
        scratch_types=[
            pltpu.VMEM((_W,), jnp.int32),
            pltpu.VMEM((_W,), jnp.int32),
            pltpu.VMEM((_W, EMBED_DIM), jnp.float32),
            pltpu.VMEM((_W, EMBED_DIM), jnp.float32),
            pltpu.VMEM((_W * EMBED_DIM,), jnp.float32),
            pltpu.VMEM((_W * EMBED_DIM,), jnp.float32),
            pltpu.SemaphoreType.DMA,
            pltpu.SemaphoreType.DMA,
            pltpu.SemaphoreType.DMA,
            pltpu.SemaphoreType.DMA,
            pltpu.SemaphoreType.DMA,
            pltpu.SemaphoreType.DMA,
        ],
        compiler_params=pltpu.CompilerParams(
            use_tc_tiling_on_sc=False, needs_layout_passes=False),
    )
    o3 = gather(enc4, table)
    dec = (o3.reshape(SEQ, 4, 128, 8, 128)
              .transpose(2, 4, 0, 1, 3)
              .reshape(BATCH, SEQ, EMBED_DIM))
    return (dec, enc_mask)


# speedup vs baseline: 1.6968x; 1.0004x over previous
import jax
import jax.numpy as jnp
from jax import lax
from jax.experimental import pallas as pl
from jax.experimental.pallas import tpu as pltpu
from jax.experimental.pallas import tpu_sc as plsc

VOCAB = 1000000
EMBED_DIM = 32
BATCH = 16384
SEQ = 200

_NW = 32
_W = BATCH // _NW


def _body(enc4, table, out3,
          idx0, idx1, rows0, rows1, ob0, ob1,
          isem0, isem1, gsem0, gsem1, osem0, osem1):
    w = lax.axis_index("s") * 2 + lax.axis_index("c")
    k0 = w * (_W * 8)
    idx_v = (idx0, idx1)
    rows_v = (rows0, rows1)
    obuf = (ob0, ob1)
    isem = (isem0, isem1)
    gsem = (gsem0, gsem1)
    osem = (osem0, osem1)
    iota16 = lax.iota(jnp.int32, 16)

    def idx_start(s, b):
        for c2 in range(4):
            pltpu.async_copy(
                enc4.at[s >> 3, 4 * w + c2, s & 7],
                idx_v[b].at[pl.ds(128 * c2, 128)], isem[b])

    def idx_wait(b):
        for c2 in range(4):
            pltpu.make_async_copy(
                enc4.at[0, 0, 0],
                idx_v[b].at[pl.ds(128 * c2, 128)], isem[b]).wait()

    def gather_start(b):
        pltpu.async_copy(table.at[idx_v[b]], rows_v[b], gsem[b])

    def gather_wait(b):
        pltpu.make_async_copy(table.at[idx_v[b]], rows_v[b], gsem[b]).wait()

    def out_start(s, b):
        for r2 in range(4):
            pltpu.async_copy(
                obuf[b].at[pl.ds(r2 * 4096, 4096)],
                out3.at[s, r2, pl.ds(k0, 4096)], osem[b])

    def out_wait(b):
        for r2 in range(4):
            pltpu.make_async_copy(
                obuf[b].at[pl.ds(r2 * 4096, 4096)],
                out3.at[0, r2, pl.ds(k0, 4096)], osem[b]).wait()

    colv = [iota16 ^ k for k in range(16)]
    sctr = [((iota16 ^ k) >> 3) * 4096 + ((iota16 ^ k) & 7) * 128 + iota16
            for k in range(16)]

    def transpose(b):
        rows = rows_v[b]
        ob = obuf[b]

        @plsc.parallel_loop(0, _W // 16, unroll=2)
        def _jb_body(jb):
            rowv = iota16 + jb * 16
            base_s = (jb >> 3) * 1024 + (jb & 7) * 16
            for eb in range(2):
                for k in range(16):
                    v = plsc.load_gather(rows, [rowv, colv[k] + 16 * eb])
                    plsc.store_scatter(
                        ob, [sctr[k] + (base_s + 8192 * eb)], v)

    def step(s, b):
        bp = 1 - b

        @pl.when(s + 1 < SEQ)
        def _next_gather():
            idx_wait(bp)
            gather_start(bp)

        gather_wait(b)

        @pl.when(s + 2 < SEQ)
        def _prefetch_idx():
            idx_start(s + 2, b)

        @pl.when(s >= 2)
        def _reclaim_obuf():
            out_wait(b)

        transpose(b)
        out_start(s, b)

    idx_start(0, 0)
    idx_start(1, 1)
    idx_wait(0)
    gather_start(0)

    def pair(p, carry):
        step(2 * p, 0)
        step(2 * p + 1, 1)
        return carry

    lax.fori_loop(0, SEQ // 2, pair, 0)

    out_wait(0)
    out_wait(1)


@jax.jit
def kernel(enc, enc_mask, table):
    enc4 = enc.T.reshape(25, 8, 128, 128).transpose(0, 2, 1, 3)
    mesh = plsc.VectorSubcoreMesh(core_axis_name="c", subcore_axis_name="s")
    gather = pl.kernel(
        _body,
        mesh=mesh,
        out_type=jax.ShapeDtypeStruct((SEQ, 4, BATCH * 8), jnp.float32),
        scratch_types=[
            pltpu.VMEM((_W,), jnp.int32),
            pltpu.VMEM((_W,), jnp.int32),
            pltpu.VMEM((_W, EMBED_DIM), jnp.float32),
            pltpu.VMEM((_W, EMBED_DIM), jnp.float32),
            pltpu.VMEM((_W * EMBED_DIM,), jnp.float32),
            pltpu.VMEM((_W * EMBED_DIM,), jnp.float32),
            pltpu.SemaphoreType.DMA,
            pltpu.SemaphoreType.DMA,
            pltpu.SemaphoreType.DMA,
            pltpu.SemaphoreType.DMA,
            pltpu.SemaphoreType.DMA,
            pltpu.SemaphoreType.DMA,
        ],
        compiler_params=pltpu.CompilerParams(
            use_tc_tiling_on_sc=False, needs_layout_passes=False),
    )
    o3 = gather(enc4, table)
    dec = (o3.reshape(SEQ, 4, 128, 8, 128)
              .transpose(2, 4, 0, 1, 3)
              .reshape(BATCH, SEQ, EMBED_DIM))
    return (dec, enc_mask)
